# baseline (speedup 1.0000x reference)
import jax
import jax.numpy as jnp
from jax import lax
from jax.experimental import pallas as pl
from jax.experimental.pallas import tpu as pltpu
from jax.experimental.pallas import tpu_sc as plsc

B = 1024
M = 16384
L = 16
NC = 2
NS = 16
NW = NC * NS
B_SC = 448
RPW = B_SC // NW
CH = 4096
NCH = M // CH
T = RPW * NCH

BB = 64
BM = 4096
ROFF = B_SC // BB


def _sc_body(ww_hbm, fgx_hbm, rw_hbm, pu_hbm, out_hbm,
             fgw_v, ww_v, rw_v, pu_v, out_v,
             sem_in0, sem_in1, sem_out0, sem_out1):
    wid = lax.axis_index("s") * NC + lax.axis_index("c")
    base = wid * RPW
    sems_in = (sem_in0, sem_in1)
    sems_out = (sem_out0, sem_out1)

    pltpu.sync_copy(fgx_hbm.at[pl.ds(base, RPW)], fgw_v)

    def task_coords(t):
        i = t // NCH
        c = t - i * NCH
        return base + i, i, c * CH

    def start_in(t, s):
        b, _, off = task_coords(t)
        pltpu.async_copy(ww_hbm.at[b, :, pl.ds(off, CH)], ww_v.at[s],
                         sems_in[s])
        pltpu.async_copy(rw_hbm.at[b, :, pl.ds(off, CH)], rw_v.at[s],
                         sems_in[s])
        pltpu.async_copy(pu_hbm.at[b, pl.ds(off, CH)], pu_v.at[s],
                         sems_in[s])

    def wait_in(s):
        pltpu.make_async_copy(ww_hbm.at[0, :, pl.ds(0, CH)], ww_v.at[s],
                              sems_in[s]).wait()
        pltpu.make_async_copy(rw_hbm.at[0, :, pl.ds(0, CH)], rw_v.at[s],
                              sems_in[s]).wait()
        pltpu.make_async_copy(pu_hbm.at[0, pl.ds(0, CH)], pu_v.at[s],
                              sems_in[s]).wait()

    def wait_out(s):
        pltpu.make_async_copy(out_v.at[s], out_hbm.at[0, pl.ds(0, CH)],
                              sems_out[s]).wait()

    def compute(t, s):
        _, i, _ = task_coords(t)
        fg0 = fgw_v[i, 0, :]
        fg1 = fgw_v[i, 1, :]
        fg2 = fgw_v[i, 2, :]
        fg3 = fgw_v[i, 3, :]

        @plsc.parallel_loop(0, CH, step=L, unroll=8)
        def vec_body(k):
            sl = pl.ds(k, L)
            w0 = ww_v[s, 0, sl]
            w1 = ww_v[s, 1, sl]
            ww = 1.0 - (1.0 - w0) * (1.0 - w1)
            p = pu_v[s, sl]
            usage = p + (1.0 - p) * ww
            phi = (1.0 - fg0 * rw_v[s, 0, sl]) * (1.0 - fg1 * rw_v[s, 1, sl])
            phi = phi * (1.0 - fg2 * rw_v[s, 2, sl]) * (1.0 - fg3 * rw_v[s, 3, sl])
            res = usage * phi
            out_v[s, sl] = jnp.minimum(jnp.maximum(res, 0.0), 1.0)

    def start_out(t, s):
        b, _, off = task_coords(t)
        pltpu.async_copy(out_v.at[s], out_hbm.at[b, pl.ds(off, CH)],
                         sems_out[s])

    start_in(0, 0)

    def pair_body(g, carry):
        t0 = g * 2
        for d in range(2):
            t = t0 + d

            @pl.when(t + 1 < T)
            def _():
                start_in(t + 1, 1 - d)

            @pl.when(t >= 2)
            def _():
                wait_out(d)

            wait_in(d)
            compute(t, d)
            start_out(t, d)
        return carry

    lax.fori_loop(0, T // 2, pair_body, 0)
    wait_out(0)
    wait_out(1)


def _tc_body(fg_ref, ww_ref, rw_ref, pu_ref, out_ref):
    w0 = ww_ref[:, 0, :]
    w1 = ww_ref[:, 1, :]
    ww = 1.0 - (1.0 - w0) * (1.0 - w1)
    pu = pu_ref[...]
    usage = pu + (1.0 - pu) * ww
    fg = fg_ref[...]
    phi = 1.0 - fg[:, 0][:, None] * rw_ref[:, 0, :]
    for r in range(1, 4):
        phi = phi * (1.0 - fg[:, r][:, None] * rw_ref[:, r, :])
    out_ref[...] = jnp.clip(usage * phi, 0.0, 1.0)


def kernel(write_weights, free_gate, read_weights, prev_usage):
    fgx = jnp.broadcast_to(free_gate[:, :, None], (B, 4, L))
    mesh = plsc.VectorSubcoreMesh(core_axis_name="c", subcore_axis_name="s")
    out_sc = pl.kernel(
        _sc_body,
        out_type=jax.ShapeDtypeStruct((B_SC, M), jnp.float32),
        mesh=mesh,
        scratch_types=[
            pltpu.VMEM((RPW, 4, L), jnp.float32),
            pltpu.VMEM((2, 2, CH), jnp.float32),
            pltpu.VMEM((2, 4, CH), jnp.float32),
            pltpu.VMEM((2, CH), jnp.float32),
            pltpu.VMEM((2, CH), jnp.float32),
            pltpu.SemaphoreType.DMA,
            pltpu.SemaphoreType.DMA,
            pltpu.SemaphoreType.DMA,
            pltpu.SemaphoreType.DMA,
        ],
    )(write_weights, fgx, read_weights, prev_usage)

    grid = ((B - B_SC) // BB, M // BM)
    out_tc = pl.pallas_call(
        _tc_body,
        grid=grid,
        in_specs=[
            pl.BlockSpec((BB, 4), lambda i, j: (i + ROFF, 0)),
            pl.BlockSpec((BB, 2, BM), lambda i, j: (i + ROFF, 0, j)),
            pl.BlockSpec((BB, 4, BM), lambda i, j: (i + ROFF, 0, j)),
            pl.BlockSpec((BB, BM), lambda i, j: (i + ROFF, j)),
        ],
        out_specs=pl.BlockSpec((BB, BM), lambda i, j: (i, j)),
        out_shape=jax.ShapeDtypeStruct((B - B_SC, M), jnp.float32),
        compiler_params=pltpu.CompilerParams(
            dimension_semantics=("arbitrary", "arbitrary"),
        ),
    )(free_gate, write_weights, read_weights, prev_usage)

    return jnp.concatenate([out_sc, out_tc], axis=0)


# speedup vs baseline: 1.1720x; 1.1720x over previous
import jax
import jax.numpy as jnp
from jax import lax
from jax.experimental import pallas as pl
from jax.experimental.pallas import tpu as pltpu
from jax.experimental.pallas import tpu_sc as plsc

B = 1024
M = 16384
L = 16
NC = 2
NS = 16
NW = NC * NS
RPW = B // NW
CH = 2048
NCH = M // CH
T = RPW * NCH
S = 4
AHEAD = S - 1


def _sc_body(ww_hbm, fgx_hbm, rw_hbm, pu_hbm, out_hbm,
             fgw_v, ww_v, rw_v, pu_v, out_v,
             si0, si1, si2, si3, so0, so1, so2, so3):
    wid = lax.axis_index("s") * NC + lax.axis_index("c")
    base = wid * RPW
    sems_in = (si0, si1, si2, si3)
    sems_out = (so0, so1, so2, so3)

    pltpu.sync_copy(fgx_hbm.at[pl.ds(base, RPW)], fgw_v)

    def task_coords(t):
        i = t // NCH
        c = t - i * NCH
        return base + i, i, c * CH

    def start_in(t, s):
        b, _, off = task_coords(t)
        pltpu.async_copy(ww_hbm.at[b, :, pl.ds(off, CH)], ww_v.at[s],
                         sems_in[s])
        pltpu.async_copy(rw_hbm.at[b, :, pl.ds(off, CH)], rw_v.at[s],
                         sems_in[s])
        pltpu.async_copy(pu_hbm.at[b, pl.ds(off, CH)], pu_v.at[s],
                         sems_in[s])

    def wait_in(s):
        pltpu.make_async_copy(ww_hbm.at[0, :, pl.ds(0, CH)], ww_v.at[s],
                              sems_in[s]).wait()
        pltpu.make_async_copy(rw_hbm.at[0, :, pl.ds(0, CH)], rw_v.at[s],
                              sems_in[s]).wait()
        pltpu.make_async_copy(pu_hbm.at[0, pl.ds(0, CH)], pu_v.at[s],
                              sems_in[s]).wait()

    def wait_out(s):
        pltpu.make_async_copy(out_v.at[s], out_hbm.at[0, pl.ds(0, CH)],
                              sems_out[s]).wait()

    def compute(t, s):
        _, i, _ = task_coords(t)
        fg0 = fgw_v[i, 0, :]
        fg1 = fgw_v[i, 1, :]
        fg2 = fgw_v[i, 2, :]
        fg3 = fgw_v[i, 3, :]

        @plsc.parallel_loop(0, CH, step=L, unroll=8)
        def vec_body(k):
            sl = pl.ds(k, L)
            w0 = ww_v[s, 0, sl]
            w1 = ww_v[s, 1, sl]
            ww = 1.0 - (1.0 - w0) * (1.0 - w1)
            p = pu_v[s, sl]
            usage = p + (1.0 - p) * ww
            phi = (1.0 - fg0 * rw_v[s, 0, sl]) * (1.0 - fg1 * rw_v[s, 1, sl])
            phi = phi * (1.0 - fg2 * rw_v[s, 2, sl]) * (1.0 - fg3 * rw_v[s, 3, sl])
            res = usage * phi
            out_v[s, sl] = jnp.minimum(jnp.maximum(res, 0.0), 1.0)

    def start_out(t, s):
        b, _, off = task_coords(t)
        pltpu.async_copy(out_v.at[s], out_hbm.at[b, pl.ds(off, CH)],
                         sems_out[s])

    for t in range(AHEAD):
        start_in(t, t)

    def group_body(g, carry):
        t0 = g * S
        for d in range(S):
            t = t0 + d

            @pl.when(t + AHEAD < T)
            def _():
                start_in(t + AHEAD, (d + AHEAD) % S)

            @pl.when(t >= S)
            def _():
                wait_out(d)

            wait_in(d)
            compute(t, d)
            start_out(t, d)
        return carry

    lax.fori_loop(0, T // S, group_body, 0)
    for s in range(S):
        wait_out(s)


def kernel(write_weights, free_gate, read_weights, prev_usage):
    fgx = jnp.broadcast_to(free_gate[:, :, None], (B, 4, L))
    mesh = plsc.VectorSubcoreMesh(core_axis_name="c", subcore_axis_name="s")
    return pl.kernel(
        _sc_body,
        out_type=jax.ShapeDtypeStruct((B, M), jnp.float32),
        mesh=mesh,
        scratch_types=[
            pltpu.VMEM((RPW, 4, L), jnp.float32),
            pltpu.VMEM((S, 2, CH), jnp.float32),
            pltpu.VMEM((S, 4, CH), jnp.float32),
            pltpu.VMEM((S, CH), jnp.float32),
            pltpu.VMEM((S, CH), jnp.float32),
            pltpu.SemaphoreType.DMA,
            pltpu.SemaphoreType.DMA,
            pltpu.SemaphoreType.DMA,
            pltpu.SemaphoreType.DMA,
            pltpu.SemaphoreType.DMA,
            pltpu.SemaphoreType.DMA,
            pltpu.SemaphoreType.DMA,
            pltpu.SemaphoreType.DMA,
        ],
    )(write_weights, fgx, read_weights, prev_usage)
